# Initial kernel scaffold; baseline (speedup 1.0000x reference)
import jax
import jax.numpy as jnp
from jax.experimental import pallas as pl


def kernel(x, edge_index, weight, w_ih, w_hh, b_ih, b_hh):
    raise NotImplementedError("write your pallas kernel here")


# speedup vs baseline: 4.2722x; 4.2722x over previous
import functools

import jax
import jax.numpy as jnp
from jax import lax
from jax.experimental import pallas as pl
from jax.experimental.pallas import tpu as pltpu
from jax.experimental.pallas import tpu_sc as plsc

NC = 2
NS = 16
NW = NC * NS
C = 128


def _matmul_body(x_ref, w_ref, o_ref):
    o_ref[...] = jnp.dot(x_ref[...], w_ref[...],
                         preferred_element_type=jnp.float32)


def _gru_body(x_ref, p0_ref, p1_ref, wih_ref, whh_ref, bi_ref, bh_ref, o_ref):
    d = x_ref.shape[1]
    xb = x_ref[...]
    agg = p0_ref[...] + p1_ref[...]
    gi = jnp.dot(agg, wih_ref[...], preferred_element_type=jnp.float32) + bi_ref[...]
    gh = jnp.dot(xb, whh_ref[...], preferred_element_type=jnp.float32) + bh_ref[...]
    i_r, i_z, i_n = gi[:, :d], gi[:, d:2 * d], gi[:, 2 * d:]
    h_r, h_z, h_n = gh[:, :d], gh[:, d:2 * d], gh[:, 2 * d:]
    r = jax.nn.sigmoid(i_r + h_r)
    z = jax.nn.sigmoid(i_z + h_z)
    n = jnp.tanh(i_n + r * h_n)
    h_new = (1.0 - z) * n + z * xb
    o_ref[...] = xb + jnp.maximum(h_new, 0.0)


def _make_scatter_kernel(n_agg, d, chunks, rows_per_tile):
    mesh = plsc.VectorSubcoreMesh(core_axis_name="c", subcore_axis_name="s",
                                  num_cores=NC, num_subcores=NS)

    @functools.partial(
        pl.kernel,
        out_type=jax.ShapeDtypeStruct((NC, n_agg, d), jnp.float32),
        mesh=mesh,
        scratch_types=[
            pltpu.VMEM_SHARED((n_agg, d), jnp.float32),
            pltpu.VMEM((chunks, C), jnp.int32),
            pltpu.VMEM((chunks, C), jnp.int32),
            pltpu.VMEM((C, d), jnp.float32),
            pltpu.SemaphoreType.DMA,
        ],
    )
    def scatter_kernel(m_hbm, src_hbm, dst_hbm, zeros_hbm, out_hbm,
                       agg_sp, src_v, dst_v, rows_v, sem):
        c = lax.axis_index("c")
        s = lax.axis_index("s")
        base = s * rows_per_tile
        pltpu.sync_copy(zeros_hbm.at[pl.ds(base, rows_per_tile)],
                        agg_sp.at[pl.ds(base, rows_per_tile)])
        pltpu.sync_copy(src_hbm.at[c, s], src_v)
        pltpu.sync_copy(dst_hbm.at[c, s], dst_v)
        plsc.subcore_barrier()

        def body(j, carry):
            pltpu.async_copy(m_hbm.at[src_v.at[j]], rows_v, sem).wait()
            pltpu.sync_copy(rows_v, agg_sp.at[dst_v.at[j]], add=True)
            return carry

        lax.fori_loop(0, chunks, body, 0)
        plsc.subcore_barrier()
        pltpu.sync_copy(agg_sp.at[pl.ds(base, rows_per_tile)],
                        out_hbm.at[c, pl.ds(base, rows_per_tile)])

    return scatter_kernel


def kernel(x, edge_index, weight, w_ih, w_hh, b_ih, b_hh):
    n, d = x.shape
    e = edge_index.shape[1]

    per_w = -(-e // NW)
    chunks = -(-per_w // C)
    e_pad = NW * chunks * C
    dummy_dst = n
    n_agg = -(-(n + 1) // (NS * 8)) * (NS * 8)
    rows_per_tile = n_agg // NS

    src = jnp.concatenate(
        [edge_index[0], jnp.zeros((e_pad - e,), jnp.int32)]).reshape(
            NC, NS, chunks, C)
    dst = jnp.concatenate(
        [edge_index[1], jnp.full((e_pad - e,), dummy_dst, jnp.int32)]).reshape(
            NC, NS, chunks, C)
    zeros_hbm = jnp.zeros((n_agg, d), jnp.float32)

    br = 2000
    m = pl.pallas_call(
        _matmul_body,
        grid=(n // br,),
        in_specs=[pl.BlockSpec((br, d), lambda i: (i, 0)),
                  pl.BlockSpec((d, d), lambda i: (0, 0))],
        out_specs=pl.BlockSpec((br, d), lambda i: (i, 0)),
        out_shape=jax.ShapeDtypeStruct((n, d), jnp.float32),
    )(x, weight[0])

    partial = _make_scatter_kernel(n_agg, d, chunks, rows_per_tile)(
        m, src, dst, zeros_hbm)

    out = pl.pallas_call(
        _gru_body,
        grid=(n // br,),
        in_specs=[
            pl.BlockSpec((br, d), lambda i: (i, 0)),
            pl.BlockSpec((br, d), lambda i: (i, 0)),
            pl.BlockSpec((br, d), lambda i: (i, 0)),
            pl.BlockSpec((d, 3 * d), lambda i: (0, 0)),
            pl.BlockSpec((d, 3 * d), lambda i: (0, 0)),
            pl.BlockSpec((1, 3 * d), lambda i: (0, 0)),
            pl.BlockSpec((1, 3 * d), lambda i: (0, 0)),
        ],
        out_specs=pl.BlockSpec((br, d), lambda i: (i, 0)),
        out_shape=jax.ShapeDtypeStruct((n, d), jnp.float32),
    )(x, partial[0, :n], partial[1, :n], w_ih.T, w_hh.T,
      b_ih.reshape(1, -1), b_hh.reshape(1, -1))

    return out
